# Initial kernel scaffold; baseline (speedup 1.0000x reference)
import jax
import jax.numpy as jnp
from jax.experimental import pallas as pl


def kernel(x, edge_index, pseudo, W1, r1, b1, W2, r2, b2, W3, r3, b3, W4, r4, b4, W5, r5, b5, W6, r6, b6, lw1, lb1, lw2, lb2):
    raise NotImplementedError("write your pallas kernel here")


# speedup vs baseline: 5.7893x; 5.7893x over previous
import functools

import jax
import jax.numpy as jnp
from jax import lax
from jax.experimental import pallas as pl
from jax.experimental.pallas import tpu as pltpu
from jax.experimental.pallas import tpu_sc as plsc

KS = 5
KKK = KS ** 3
NNODE = 6890
NPAD = 6912
NW = 32
CHUNK = 128
RPT = NPAD // 16
NCLS = 6890

f32 = jnp.float32
i32 = jnp.int32


@functools.lru_cache(maxsize=None)
def _make_edge_pass(Co, with_deg, e_pad, cpw):
    epw = e_pad // NW

    mesh = plsc.VectorSubcoreMesh(core_axis_name="c", subcore_axis_name="s")

    if with_deg:
        out_type = (jax.ShapeDtypeStruct((2, NPAD, Co), f32),
                    jax.ShapeDtypeStruct((2, NPAD, 16), f32))
    else:
        out_type = jax.ShapeDtypeStruct((2, NPAD, Co), f32)

    scratch = [pltpu.VMEM_SHARED((NPAD, Co), f32)]
    if with_deg:
        scratch.append(pltpu.VMEM_SHARED((NPAD, 16), f32))
    scratch += [
        pltpu.VMEM((8 * CHUNK, Co), f32),
        pltpu.VMEM((CHUNK, Co), f32),
        pltpu.VMEM((8 * CHUNK,), i32),
        pltpu.VMEM((8 * CHUNK + 16,), f32),
        pltpu.VMEM((CHUNK,), i32),
        pltpu.VMEM((CHUNK,), i32),
        pltpu.VMEM((3 * CHUNK,), f32),
    ]
    if with_deg:
        scratch.append(pltpu.VMEM((CHUNK, 16), f32))
    scratch.append(pltpu.SemaphoreType.DMA)

    def body(*refs):
        it = iter(refs)
        z_hbm = next(it)
        p_hbm = next(it)
        src_hbm = next(it)
        dst_hbm = next(it)
        zac_hbm = next(it)
        if with_deg:
            zdg_hbm = next(it)
            one_hbm = next(it)
        out_hbm = next(it)
        if with_deg:
            deg_hbm = next(it)
        acc = next(it)
        if with_deg:
            accd = next(it)
        rows = next(it)
        msg = next(it)
        gid = next(it)
        bw = next(it)
        srcv = next(it)
        dstv = next(it)
        pv = next(it)
        if with_deg:
            onesb = next(it)
        sem = next(it)

        c = lax.axis_index("c")
        sid = lax.axis_index("s")
        wid = sid * 2 + c
        r0 = sid * RPT

        pltpu.sync_copy(zac_hbm.at[pl.ds(r0, RPT), :], acc.at[pl.ds(r0, RPT), :])
        if with_deg:
            pltpu.sync_copy(zdg_hbm.at[pl.ds(r0, RPT), :],
                            accd.at[pl.ds(r0, RPT), :])
            pltpu.sync_copy(one_hbm, onesb)
        plsc.subcore_barrier()

        lane = lax.broadcasted_iota(i32, (16,), 0)

        def chunk_body(t, carry):
            base = wid * epw + t * CHUNK
            pltpu.sync_copy(src_hbm.at[pl.ds(base, CHUNK)], srcv)
            pltpu.sync_copy(dst_hbm.at[pl.ds(base, CHUNK)], dstv)
            for d in range(3):
                pltpu.sync_copy(p_hbm.at[pl.ds(d * e_pad + base, CHUNK)],
                                pv.at[pl.ds(d * CHUNK, CHUNK)])
            for g in range(CHUNK // 16):
                sg = srcv[pl.ds(g * 16, 16)]
                fr = []
                bo = []
                for d in range(3):
                    v = pv[pl.ds(d * CHUNK + g * 16, 16)]
                    bi = v.astype(i32)
                    fr.append(v - bi.astype(f32))
                    bo.append(bi)
                gbase = sg * KKK
                for s in range(8):
                    b = None
                    idx = None
                    stride = 1
                    for d in range(3):
                        o = (s >> d) & 1
                        f = fr[d] if o else (1.0 - fr[d])
                        b = f if b is None else b * f
                        kd = jnp.minimum(bo[d] + o, KS - 1)
                        term = kd * stride
                        idx = term if idx is None else idx + term
                        stride *= KS
                    gid[pl.ds(s * CHUNK + g * 16, 16)] = gbase + idx
                    bw[pl.ds(s * CHUNK + g * 16, 16)] = b
            cps = [pltpu.async_copy(z_hbm.at[gid.at[pl.ds(s * CHUNK, CHUNK)]],
                                    rows.at[pl.ds(s * CHUNK, CHUNK), :], sem)
                   for s in range(8)]
            for cp in cps:
                cp.wait()

            def edge_body(e, cc2):
                bs = [bw[pl.ds(s * CHUNK + e, 16)][0] for s in range(8)]
                for ccc in range(Co // 16):
                    accv = jnp.zeros((16,), f32)
                    for s in range(8):
                        rv = rows[s * CHUNK + e, pl.ds(ccc * 16, 16)]
                        accv = accv + rv * bs[s]
                    msg[e, pl.ds(ccc * 16, 16)] = accv
                return cc2
            lax.fori_loop(0, CHUNK, edge_body, 0)

            pltpu.sync_copy(msg, acc.at[dstv], add=True)
            if with_deg:
                pltpu.sync_copy(onesb, accd.at[dstv], add=True)
            return carry

        lax.fori_loop(0, cpw, chunk_body, 0)
        plsc.subcore_barrier()
        pltpu.sync_copy(acc.at[pl.ds(r0, RPT), :],
                        out_hbm.at[c, pl.ds(r0, RPT), :])
        if with_deg:
            pltpu.sync_copy(accd.at[pl.ds(r0, RPT), :],
                            deg_hbm.at[c, pl.ds(r0, RPT), :])

    return pl.kernel(body, out_type=out_type, mesh=mesh,
                     scratch_types=scratch,
                     compiler_params=pltpu.CompilerParams(
                         use_tc_tiling_on_sc=False))


@functools.lru_cache(maxsize=None)
def _make_mm(Cin, KCo):
    def kfn(h_ref, w_ref, o_ref):
        if Cin == 1:
            o_ref[...] = h_ref[...] * w_ref[...]
        else:
            o_ref[...] = jnp.dot(h_ref[...], w_ref[...],
                                 preferred_element_type=f32)

    return pl.pallas_call(
        kfn, grid=(NPAD // 128,),
        in_specs=[pl.BlockSpec((128, Cin), lambda i: (i, 0)),
                  pl.BlockSpec((Cin, KCo), lambda i: (0, 0))],
        out_specs=pl.BlockSpec((128, KCo), lambda i: (i, 0)),
        out_shape=jax.ShapeDtypeStruct((NPAD, KCo), f32))


@functools.lru_cache(maxsize=None)
def _make_epi(Cin, Co):
    def kfn(p_ref, d_ref, h_ref, r_ref, b_ref, o_ref):
        psum = p_ref[0] + p_ref[1]
        deg = d_ref[0, :, 0:1] + d_ref[1, :, 0:1]
        if Cin == 1:
            xr = h_ref[...] * r_ref[...]
        else:
            xr = jnp.dot(h_ref[...], r_ref[...], preferred_element_type=f32)
        val = psum / jnp.maximum(deg, 1.0) + xr + b_ref[...]
        o_ref[...] = jnp.where(val > 0, val,
                               jnp.exp(jnp.minimum(val, 0.0)) - 1.0)

    return pl.pallas_call(
        kfn, grid=(NPAD // 128,),
        in_specs=[pl.BlockSpec((2, 128, Co), lambda i: (0, i, 0)),
                  pl.BlockSpec((2, 128, 16), lambda i: (0, i, 0)),
                  pl.BlockSpec((128, Cin), lambda i: (i, 0)),
                  pl.BlockSpec((Cin, Co), lambda i: (0, 0)),
                  pl.BlockSpec((1, Co), lambda i: (0, 0))],
        out_specs=pl.BlockSpec((128, Co), lambda i: (i, 0)),
        out_shape=jax.ShapeDtypeStruct((NPAD, Co), f32))


def _head_kernel(h_ref, w1_ref, b1_ref, w2_ref, b2_ref, o_ref):
    a = jnp.dot(h_ref[...], w1_ref[...], preferred_element_type=f32) + b1_ref[...]
    a = jnp.where(a > 0, a, jnp.exp(jnp.minimum(a, 0.0)) - 1.0)
    logits = jnp.dot(a, w2_ref[...], preferred_element_type=f32) + b2_ref[...]
    m = jnp.max(logits, axis=1, keepdims=True)
    lse = jnp.log(jnp.sum(jnp.exp(logits - m), axis=1, keepdims=True)) + m
    o_ref[...] = logits - lse


_head = pl.pallas_call(
    _head_kernel, grid=(NPAD // 128,),
    in_specs=[pl.BlockSpec((128, 64), lambda i: (i, 0)),
              pl.BlockSpec((64, 256), lambda i: (0, 0)),
              pl.BlockSpec((1, 256), lambda i: (0, 0)),
              pl.BlockSpec((256, NCLS), lambda i: (0, 0)),
              pl.BlockSpec((1, NCLS), lambda i: (0, 0))],
    out_specs=pl.BlockSpec((128, NCLS), lambda i: (i, 0)),
    out_shape=jax.ShapeDtypeStruct((NNODE, NCLS), f32))


def kernel(x, edge_index, pseudo, W1, r1, b1, W2, r2, b2, W3, r3, b3,
           W4, r4, b4, W5, r5, b5, W6, r6, b6, lw1, lb1, lw2, lb2):
    E = pseudo.shape[0]
    e_pad = -(-E // (NW * CHUNK)) * (NW * CHUNK)
    cpw = e_pad // (NW * CHUNK)
    pad_e = e_pad - E

    src = edge_index[0].astype(i32)
    dst = edge_index[1].astype(i32)
    srcp = jnp.pad(src, (0, pad_e))
    dstp = jnp.pad(dst, (0, pad_e), constant_values=NNODE)
    pq = (pseudo.astype(f32) * (KS - 1)).T
    pflat = jnp.pad(pq, ((0, 0), (0, pad_e))).reshape(-1)
    h = jnp.pad(x.astype(f32), ((0, NPAD - NNODE), (0, 0)))

    zdeg = jnp.zeros((NPAD, 16), f32)
    ones16 = jnp.ones((CHUNK, 16), f32)

    deg2 = None
    for li, (W, r, b) in enumerate([(W1, r1, b1), (W2, r2, b2), (W3, r3, b3),
                                    (W4, r4, b4), (W5, r5, b5), (W6, r6, b6)]):
        Cin, Co = W.shape[1], W.shape[2]
        W2d = jnp.transpose(W, (1, 0, 2)).reshape(Cin, KKK * Co)
        z = _make_mm(Cin, KKK * Co)(h, W2d)
        z2 = z.reshape(NPAD * KKK, Co)
        zacc = jnp.zeros((NPAD, Co), f32)
        ep = _make_edge_pass(Co, li == 0, e_pad, cpw)
        if li == 0:
            parts, deg2 = ep(z2, pflat, srcp, dstp, zacc, zdeg, ones16)
        else:
            parts = ep(z2, pflat, srcp, dstp, zacc)
        h = _make_epi(Cin, Co)(parts, deg2, h, r, jnp.reshape(b, (1, Co)))

    return _head(h, lw1, jnp.reshape(lb1, (1, 256)), lw2,
                 jnp.reshape(lb2, (1, NCLS)))


# speedup vs baseline: 7.1121x; 1.2285x over previous
import functools

import jax
import jax.numpy as jnp
from jax import lax
from jax.experimental import pallas as pl
from jax.experimental.pallas import tpu as pltpu
from jax.experimental.pallas import tpu_sc as plsc

KS = 5
KKK = KS ** 3
NNODE = 6890
NPAD = 6912
NW = 32
CHUNK = 32
RPT = NPAD // 16
NCLS = 6890

f32 = jnp.float32
i32 = jnp.int32


@functools.lru_cache(maxsize=None)
def _make_edge_pass(Co, with_deg, e_pad):
    epw = e_pad // NW
    nch = epw // CHUNK
    assert nch % 2 == 0
    ngrp = epw // 16
    rpc = 8 * CHUNK

    mesh = plsc.VectorSubcoreMesh(core_axis_name="c", subcore_axis_name="s")

    if with_deg:
        out_type = (jax.ShapeDtypeStruct((2, NPAD, Co), f32),
                    jax.ShapeDtypeStruct((2, NPAD, 16), f32))
    else:
        out_type = jax.ShapeDtypeStruct((2, NPAD, Co), f32)

    scratch = [pltpu.VMEM_SHARED((NPAD, Co), f32)]
    if with_deg:
        scratch.append(pltpu.VMEM_SHARED((NPAD, 16), f32))
    scratch += [
        pltpu.VMEM((rpc, Co), f32),
        pltpu.VMEM((rpc, Co), f32),
        pltpu.VMEM((CHUNK, Co), f32),
        pltpu.VMEM((CHUNK, Co), f32),
        pltpu.VMEM((CHUNK,), i32),
        pltpu.VMEM((CHUNK,), i32),
        pltpu.VMEM((8 * epw,), i32),
        pltpu.VMEM((8 * epw,), f32),
        pltpu.VMEM((epw,), i32),
        pltpu.VMEM((epw,), i32),
        pltpu.VMEM((3 * epw,), f32),
    ]
    if with_deg:
        scratch.append(pltpu.VMEM((CHUNK, 16), f32))
    scratch += [pltpu.SemaphoreType.DMA, pltpu.SemaphoreType.DMA]

    def body(*refs):
        it = iter(refs)
        z_hbm = next(it)
        p_hbm = next(it)
        src_hbm = next(it)
        dst_hbm = next(it)
        zac_hbm = next(it)
        if with_deg:
            zdg_hbm = next(it)
            one_hbm = next(it)
        out_hbm = next(it)
        if with_deg:
            deg_hbm = next(it)
        acc = next(it)
        if with_deg:
            accd = next(it)
        rowsA = next(it)
        rowsB = next(it)
        msgA = next(it)
        msgB = next(it)
        dstA = next(it)
        dstB = next(it)
        gid = next(it)
        bw = next(it)
        srcw = next(it)
        dstw = next(it)
        pvw = next(it)
        if with_deg:
            onesb = next(it)
        semA = next(it)
        semB = next(it)

        c = lax.axis_index("c")
        sid = lax.axis_index("s")
        wid = sid * 2 + c
        r0 = sid * RPT
        w0 = wid * epw

        pltpu.sync_copy(zac_hbm.at[pl.ds(r0, RPT), :], acc.at[pl.ds(r0, RPT), :])
        if with_deg:
            pltpu.sync_copy(zdg_hbm.at[pl.ds(r0, RPT), :],
                            accd.at[pl.ds(r0, RPT), :])
            pltpu.sync_copy(one_hbm, onesb)
        pltpu.sync_copy(src_hbm.at[pl.ds(w0, epw)], srcw)
        pltpu.sync_copy(dst_hbm.at[pl.ds(w0, epw)], dstw)
        for d in range(3):
            pltpu.sync_copy(p_hbm.at[pl.ds(d * e_pad + w0, epw)],
                            pvw.at[pl.ds(d * epw, epw)])
        plsc.subcore_barrier()

        gpc = CHUNK // 16

        def basis_body(g, carry):
            t = g // gpc
            gg = g % gpc
            sg = srcw[pl.ds(g * 16, 16)]
            fr = []
            bo = []
            for d in range(3):
                v = pvw[pl.ds(d * epw + g * 16, 16)]
                bi = v.astype(i32)
                fr.append(v - bi.astype(f32))
                bo.append(bi)
            gbase = sg * KKK
            pos0 = t * rpc + gg * 16
            for s in range(8):
                b = None
                idx = None
                stride = 1
                for d in range(3):
                    o = (s >> d) & 1
                    f = fr[d] if o else (1.0 - fr[d])
                    b = f if b is None else b * f
                    kd = jnp.minimum(bo[d] + o, KS - 1)
                    term = kd * stride
                    idx = term if idx is None else idx + term
                    stride *= KS
                bw[pl.ds(pos0 + s * CHUNK, 16)] = b
                gid[pl.ds(pos0 + s * CHUNK, 16)] = gbase + idx
            return carry

        lax.fori_loop(0, ngrp, basis_body, 0)

        def fire(t, rowsX, semX):
            for s in range(8):
                pltpu.async_copy(
                    z_hbm.at[gid.at[pl.ds(t * rpc + s * CHUNK, CHUNK)]],
                    rowsX.at[pl.ds(s * CHUNK, CHUNK), :], semX)

        def drain(t, rowsX, semX):
            for s in range(8):
                pltpu.make_async_copy(
                    z_hbm.at[gid.at[pl.ds(t * rpc + s * CHUNK, CHUNK)]],
                    rowsX.at[pl.ds(s * CHUNK, CHUNK), :], semX).wait()

        def reduce_chunk(t, rowsX, msgX, dstX):
            for g in range(gpc):
                dstX[pl.ds(g * 16, 16)] = dstw[pl.ds(t * CHUNK + g * 16, 16)]

            def group_body(g, carry):
                pos0 = t * rpc + g * 16
                bvecs = [bw[pl.ds(pos0 + s * CHUNK, 16)] for s in range(8)]
                rbase = g * 16
                for eg in range(16):
                    for ccc in range(Co // 16):
                        accv = None
                        for s in range(8):
                            rv = rowsX[s * CHUNK + rbase + eg,
                                       pl.ds(ccc * 16, 16)]
                            term = rv * bvecs[s][eg]
                            accv = term if accv is None else accv + term
                        msgX[rbase + eg, pl.ds(ccc * 16, 16)] = accv
                return carry

            lax.fori_loop(0, gpc, group_body, 0)

        def do_scatter(msgX, dstX):
            pltpu.sync_copy(msgX, acc.at[dstX], add=True)
            if with_deg:
                pltpu.sync_copy(onesb, accd.at[dstX], add=True)

        fire(0, rowsA, semA)
        fire(1, rowsB, semB)

        def pipe_body(t2, carry):
            tA = t2 * 2
            tB = tA + 1
            drain(tA, rowsA, semA)
            reduce_chunk(tA, rowsA, msgA, dstA)

            @pl.when(tA + 2 < nch)
            def _():
                fire(tA + 2, rowsA, semA)

            do_scatter(msgA, dstA)

            drain(tB, rowsB, semB)
            reduce_chunk(tB, rowsB, msgB, dstB)

            @pl.when(tB + 2 < nch)
            def _():
                fire(tB + 2, rowsB, semB)

            do_scatter(msgB, dstB)
            return carry

        lax.fori_loop(0, nch // 2, pipe_body, 0)

        plsc.subcore_barrier()
        pltpu.sync_copy(acc.at[pl.ds(r0, RPT), :],
                        out_hbm.at[c, pl.ds(r0, RPT), :])
        if with_deg:
            pltpu.sync_copy(accd.at[pl.ds(r0, RPT), :],
                            deg_hbm.at[c, pl.ds(r0, RPT), :])

    return pl.kernel(body, out_type=out_type, mesh=mesh,
                     scratch_types=scratch,
                     compiler_params=pltpu.CompilerParams(
                         use_tc_tiling_on_sc=False))


@functools.lru_cache(maxsize=None)
def _make_mm(Cin, KCo):
    def kfn(h_ref, w_ref, o_ref):
        if Cin == 1:
            o_ref[...] = h_ref[...] * w_ref[...]
        else:
            o_ref[...] = jnp.dot(h_ref[...], w_ref[...],
                                 preferred_element_type=f32)

    return pl.pallas_call(
        kfn, grid=(NPAD // 128,),
        in_specs=[pl.BlockSpec((128, Cin), lambda i: (i, 0)),
                  pl.BlockSpec((Cin, KCo), lambda i: (0, 0))],
        out_specs=pl.BlockSpec((128, KCo), lambda i: (i, 0)),
        out_shape=jax.ShapeDtypeStruct((NPAD, KCo), f32))


@functools.lru_cache(maxsize=None)
def _make_epi(Cin, Co):
    def kfn(p_ref, d_ref, h_ref, r_ref, b_ref, o_ref):
        psum = p_ref[0] + p_ref[1]
        deg = d_ref[0, :, 0:1] + d_ref[1, :, 0:1]
        if Cin == 1:
            xr = h_ref[...] * r_ref[...]
        else:
            xr = jnp.dot(h_ref[...], r_ref[...], preferred_element_type=f32)
        val = psum / jnp.maximum(deg, 1.0) + xr + b_ref[...]
        o_ref[...] = jnp.where(val > 0, val,
                               jnp.exp(jnp.minimum(val, 0.0)) - 1.0)

    return pl.pallas_call(
        kfn, grid=(NPAD // 128,),
        in_specs=[pl.BlockSpec((2, 128, Co), lambda i: (0, i, 0)),
                  pl.BlockSpec((2, 128, 16), lambda i: (0, i, 0)),
                  pl.BlockSpec((128, Cin), lambda i: (i, 0)),
                  pl.BlockSpec((Cin, Co), lambda i: (0, 0)),
                  pl.BlockSpec((1, Co), lambda i: (0, 0))],
        out_specs=pl.BlockSpec((128, Co), lambda i: (i, 0)),
        out_shape=jax.ShapeDtypeStruct((NPAD, Co), f32))


def _head_kernel(h_ref, w1_ref, b1_ref, w2_ref, b2_ref, o_ref):
    a = jnp.dot(h_ref[...], w1_ref[...], preferred_element_type=f32) + b1_ref[...]
    a = jnp.where(a > 0, a, jnp.exp(jnp.minimum(a, 0.0)) - 1.0)
    logits = jnp.dot(a, w2_ref[...], preferred_element_type=f32) + b2_ref[...]
    m = jnp.max(logits, axis=1, keepdims=True)
    lse = jnp.log(jnp.sum(jnp.exp(logits - m), axis=1, keepdims=True)) + m
    o_ref[...] = logits - lse


_head = pl.pallas_call(
    _head_kernel, grid=(NPAD // 128,),
    in_specs=[pl.BlockSpec((128, 64), lambda i: (i, 0)),
              pl.BlockSpec((64, 256), lambda i: (0, 0)),
              pl.BlockSpec((1, 256), lambda i: (0, 0)),
              pl.BlockSpec((256, NCLS), lambda i: (0, 0)),
              pl.BlockSpec((1, NCLS), lambda i: (0, 0))],
    out_specs=pl.BlockSpec((128, NCLS), lambda i: (i, 0)),
    out_shape=jax.ShapeDtypeStruct((NNODE, NCLS), f32))


def kernel(x, edge_index, pseudo, W1, r1, b1, W2, r2, b2, W3, r3, b3,
           W4, r4, b4, W5, r5, b5, W6, r6, b6, lw1, lb1, lw2, lb2):
    E = pseudo.shape[0]
    qe = NW * CHUNK * 2
    e_pad = -(-E // qe) * qe
    pad_e = e_pad - E

    src = edge_index[0].astype(i32)
    dst = edge_index[1].astype(i32)
    srcp = jnp.pad(src, (0, pad_e))
    dstp = jnp.pad(dst, (0, pad_e), constant_values=NNODE)
    pq = (pseudo.astype(f32) * (KS - 1)).T
    pflat = jnp.pad(pq, ((0, 0), (0, pad_e))).reshape(-1)
    h = jnp.pad(x.astype(f32), ((0, NPAD - NNODE), (0, 0)))

    zdeg = jnp.zeros((NPAD, 16), f32)
    ones16 = jnp.ones((CHUNK, 16), f32)

    deg2 = None
    for li, (W, r, b) in enumerate([(W1, r1, b1), (W2, r2, b2), (W3, r3, b3),
                                    (W4, r4, b4), (W5, r5, b5), (W6, r6, b6)]):
        Cin, Co = W.shape[1], W.shape[2]
        W2d = jnp.transpose(W, (1, 0, 2)).reshape(Cin, KKK * Co)
        z = _make_mm(Cin, KKK * Co)(h, W2d)
        z2 = z.reshape(NPAD * KKK, Co)
        zacc = jnp.zeros((NPAD, Co), f32)
        ep = _make_edge_pass(Co, li == 0, e_pad)
        if li == 0:
            parts, deg2 = ep(z2, pflat, srcp, dstp, zacc, zdeg, ones16)
        else:
            parts = ep(z2, pflat, srcp, dstp, zacc)
        h = _make_epi(Cin, Co)(parts, deg2, h, r, jnp.reshape(b, (1, Co)))

    return _head(h, lw1, jnp.reshape(lb1, (1, 256)), lw2,
                 jnp.reshape(lb2, (1, NCLS)))
